# Initial kernel scaffold; baseline (speedup 1.0000x reference)
import functools
import jax
import jax.numpy as jnp
from jax import lax
from jax.experimental import pallas as pl
from jax.experimental.pallas import tpu as pltpu
from jax.experimental.pallas import tpu_sc as plsc

_NUM_WORKERS = 32
_LANES = 16


@functools.partial(jax.jit, static_argnames=("seq_len", "num_visits"))
def _build_h(visit_boundaries, *, seq_len, num_visits):
    num_b = visit_boundaries.shape[0]
    rows_per_w = seq_len // _NUM_WORKERS
    flat_per_w = rows_per_w * num_visits
    groups = rows_per_w // _LANES
    mesh = plsc.VectorSubcoreMesh(core_axis_name="c", subcore_axis_name="s")

    @functools.partial(
        pl.kernel,
        out_type=jax.ShapeDtypeStruct((seq_len * num_visits,), jnp.float32),
        mesh=mesh,
        scratch_types=[
            pltpu.VMEM((num_b,), jnp.int32),
            pltpu.VMEM((flat_per_w,), jnp.float32),
        ],
    )
    def k(vb_hbm, out_hbm, vb, buf):
        wid = lax.axis_index("s") * 2 + lax.axis_index("c")
        pltpu.sync_copy(vb_hbm, vb)

        zeros = jnp.zeros((_LANES,), jnp.float32)

        def zero_body(i, carry):
            buf[pl.ds(pl.multiple_of(i * _LANES, _LANES), _LANES)] = zeros
            return carry

        lax.fori_loop(0, flat_per_w // _LANES, zero_body, 0)

        base = wid * rows_per_w
        lane = lax.iota(jnp.int32, _LANES)
        ones = jnp.ones((_LANES,), jnp.float32)
        for g in range(groups):
            local = g * _LANES + lane
            r = base + local
            pos = jnp.zeros((_LANES,), jnp.int32)
            s = num_b // 2
            while s >= 1:
                val = plsc.load_gather(vb, [pos + (s - 1)])
                pos = jnp.where(val <= r, pos + s, pos)
                s //= 2
            val = plsc.load_gather(vb, [pos])
            pos = jnp.where(val <= r, pos + 1, pos)
            plsc.store_scatter(buf, [local * num_visits + pos], ones)

        pltpu.sync_copy(buf, out_hbm.at[pl.ds(wid * flat_per_w, flat_per_w)])

    return k(visit_boundaries)


def kernel(X, visit_boundaries):
    seq_len = X.shape[0]
    num_visits = visit_boundaries.shape[0] + 1
    flat = _build_h(
        visit_boundaries.astype(jnp.int32), seq_len=seq_len, num_visits=num_visits
    )
    return flat.reshape(seq_len, num_visits).astype(X.dtype)


# speedup vs baseline: 7.5292x; 7.5292x over previous
import functools
import jax
import jax.numpy as jnp
from jax import lax
from jax.experimental import pallas as pl
from jax.experimental.pallas import tpu as pltpu
from jax.experimental.pallas import tpu_sc as plsc

_NUM_WORKERS = 32
_LANES = 16


@functools.partial(jax.jit, static_argnames=("seq_len", "num_visits"))
def _build_h(visit_boundaries, *, seq_len, num_visits):
    num_b = visit_boundaries.shape[0]
    rows_per_w = seq_len // _NUM_WORKERS
    flat_per_w = rows_per_w * num_visits
    groups = rows_per_w // _LANES
    mesh = plsc.VectorSubcoreMesh(core_axis_name="c", subcore_axis_name="s")

    @functools.partial(
        pl.kernel,
        out_type=jax.ShapeDtypeStruct((seq_len * num_visits,), jnp.float32),
        mesh=mesh,
        scratch_types=[
            pltpu.VMEM((num_b,), jnp.int32),
            pltpu.VMEM((flat_per_w,), jnp.float32),
        ],
        compiler_params=pltpu.CompilerParams(needs_layout_passes=False),
    )
    def k(vb_hbm, out_hbm, vb, buf):
        wid = lax.axis_index("s") * 2 + lax.axis_index("c")
        pltpu.sync_copy(vb_hbm, vb)

        zeros = jnp.zeros((_LANES,), jnp.float32)

        def zero_body(i, carry):
            buf[pl.ds(pl.multiple_of(i * _LANES, _LANES), _LANES)] = zeros
            return carry

        lax.fori_loop(0, flat_per_w // _LANES, zero_body, 0)

        base = wid * rows_per_w
        lane = lax.iota(jnp.int32, _LANES)
        ones = jnp.ones((_LANES,), jnp.float32)
        for g in range(groups):
            local = g * _LANES + lane
            r = base + local
            pos = jnp.zeros((_LANES,), jnp.int32)
            s = num_b // 2
            while s >= 1:
                val = plsc.load_gather(vb, [pos + (s - 1)])
                pos = jnp.where(val <= r, pos + s, pos)
                s //= 2
            val = plsc.load_gather(vb, [pos])
            pos = jnp.where(val <= r, pos + 1, pos)
            plsc.store_scatter(buf, [local * num_visits + pos], ones)

        pltpu.sync_copy(buf, out_hbm.at[pl.ds(wid * flat_per_w, flat_per_w)])

    return k(visit_boundaries)


def kernel(X, visit_boundaries):
    seq_len = X.shape[0]
    num_visits = visit_boundaries.shape[0] + 1
    flat = _build_h(
        visit_boundaries.astype(jnp.int32), seq_len=seq_len, num_visits=num_visits
    )
    return flat.reshape(seq_len, num_visits).astype(X.dtype)


# speedup vs baseline: 8.5718x; 1.1385x over previous
import functools
import jax
import jax.numpy as jnp
from jax import lax
from jax.experimental import pallas as pl
from jax.experimental.pallas import tpu as pltpu
from jax.experimental.pallas import tpu_sc as plsc

_NUM_WORKERS = 32
_LANES = 16


@functools.partial(jax.jit, static_argnames=("seq_len", "num_visits"))
def _build_h(visit_boundaries, *, seq_len, num_visits):
    num_b = visit_boundaries.shape[0]
    rows_per_w = seq_len // _NUM_WORKERS
    flat_per_w = rows_per_w * num_visits
    groups = rows_per_w // _LANES
    mesh = plsc.VectorSubcoreMesh(core_axis_name="c", subcore_axis_name="s")

    @functools.partial(
        pl.kernel,
        out_type=jax.ShapeDtypeStruct((seq_len * num_visits,), jnp.float32),
        mesh=mesh,
        scratch_types=[
            pltpu.VMEM((num_b,), jnp.int32),
            pltpu.VMEM((flat_per_w,), jnp.float32),
            pltpu.SemaphoreType.DMA,
        ],
        compiler_params=pltpu.CompilerParams(needs_layout_passes=False),
    )
    def k(vb_hbm, out_hbm, vb, buf, sem):
        wid = lax.axis_index("s") * 2 + lax.axis_index("c")
        vb_copy = pltpu.async_copy(vb_hbm, vb, sem)

        zeros = jnp.zeros((_LANES,), jnp.float32)
        unroll = 8
        step = unroll * _LANES

        def zero_body(i, carry):
            b = pl.multiple_of(i * step, step)
            for u in range(unroll):
                buf[pl.ds(b + u * _LANES, _LANES)] = zeros
            return carry

        lax.fori_loop(0, flat_per_w // step, zero_body, 0)
        vb_copy.wait()

        base = wid * rows_per_w
        lane = lax.iota(jnp.int32, _LANES)
        ones = jnp.ones((_LANES,), jnp.float32)
        for g in range(groups):
            local = g * _LANES + lane
            r = base + local
            pos = jnp.zeros((_LANES,), jnp.int32)
            s = num_b // 2
            while s >= 1:
                val = plsc.load_gather(vb, [pos + (s - 1)])
                pos = jnp.where(val <= r, pos + s, pos)
                s //= 2
            val = plsc.load_gather(vb, [pos])
            pos = jnp.where(val <= r, pos + 1, pos)
            plsc.store_scatter(buf, [local * num_visits + pos], ones)

        pltpu.sync_copy(buf, out_hbm.at[pl.ds(wid * flat_per_w, flat_per_w)])

    return k(visit_boundaries)


def kernel(X, visit_boundaries):
    seq_len = X.shape[0]
    num_visits = visit_boundaries.shape[0] + 1
    flat = _build_h(
        visit_boundaries.astype(jnp.int32), seq_len=seq_len, num_visits=num_visits
    )
    return flat.reshape(seq_len, num_visits).astype(X.dtype)


# speedup vs baseline: 9.7019x; 1.1318x over previous
import functools
import jax
import jax.numpy as jnp
from jax import lax
from jax.experimental import pallas as pl
from jax.experimental.pallas import tpu as pltpu
from jax.experimental.pallas import tpu_sc as plsc

_NUM_WORKERS = 32
_LANES = 16


@functools.partial(jax.jit, static_argnames=("seq_len", "num_visits"))
def _build_h(visit_boundaries, *, seq_len, num_visits):
    num_b = visit_boundaries.shape[0]
    rows_per_w = seq_len // _NUM_WORKERS
    flat_per_w = rows_per_w * num_visits
    groups = rows_per_w // _LANES
    mesh = plsc.VectorSubcoreMesh(core_axis_name="c", subcore_axis_name="s")

    @functools.partial(
        pl.kernel,
        out_type=jax.ShapeDtypeStruct((seq_len, num_visits), jnp.float32),
        mesh=mesh,
        scratch_types=[
            pltpu.VMEM((num_b,), jnp.int32),
            pltpu.VMEM((rows_per_w, num_visits), jnp.float32),
            pltpu.SemaphoreType.DMA,
        ],
        compiler_params=pltpu.CompilerParams(needs_layout_passes=False),
    )
    def k(vb_hbm, out_hbm, vb, buf, sem):
        wid = lax.axis_index("s") * 2 + lax.axis_index("c")
        vb_copy = pltpu.async_copy(vb_hbm, vb, sem)

        zeros = jnp.zeros((_LANES,), jnp.float32)
        n_full = num_visits // _LANES
        col_starts = [c * _LANES for c in range(n_full)]
        if num_visits % _LANES:
            col_starts.append(num_visits - _LANES)

        def zero_body(r, carry):
            for c in col_starts:
                buf[r, pl.ds(c, _LANES)] = zeros
            return carry

        lax.fori_loop(0, rows_per_w, zero_body, 0)
        vb_copy.wait()

        base = wid * rows_per_w
        lane = lax.iota(jnp.int32, _LANES)
        ones = jnp.ones((_LANES,), jnp.float32)
        for g in range(groups):
            local = g * _LANES + lane
            r = base + local
            pos = jnp.zeros((_LANES,), jnp.int32)
            s = num_b // 2
            while s >= 1:
                val = plsc.load_gather(vb, [pos + (s - 1)])
                pos = jnp.where(val <= r, pos + s, pos)
                s //= 2
            val = plsc.load_gather(vb, [pos])
            pos = jnp.where(val <= r, pos + 1, pos)
            plsc.store_scatter(buf, [local, pos], ones)

        pltpu.sync_copy(buf, out_hbm.at[pl.ds(wid * rows_per_w, rows_per_w)])

    return k(visit_boundaries)


def kernel(X, visit_boundaries):
    seq_len = X.shape[0]
    num_visits = visit_boundaries.shape[0] + 1
    h = _build_h(
        visit_boundaries.astype(jnp.int32), seq_len=seq_len, num_visits=num_visits
    )
    return h.astype(X.dtype)


# speedup vs baseline: 9.9277x; 1.0233x over previous
import functools
import jax
import jax.numpy as jnp
from jax import lax
from jax.experimental import pallas as pl
from jax.experimental.pallas import tpu as pltpu
from jax.experimental.pallas import tpu_sc as plsc

_NUM_WORKERS = 32
_LANES = 16


@functools.partial(jax.jit, static_argnames=("seq_len", "num_visits"))
def _build_h(visit_boundaries, *, seq_len, num_visits):
    num_b = visit_boundaries.shape[0]
    rows_per_w = seq_len // _NUM_WORKERS
    flat_per_w = rows_per_w * num_visits
    groups = rows_per_w // _LANES
    mesh = plsc.VectorSubcoreMesh(core_axis_name="c", subcore_axis_name="s")

    @functools.partial(
        pl.kernel,
        out_type=jax.ShapeDtypeStruct((seq_len, num_visits), jnp.float32),
        mesh=mesh,
        scratch_types=[
            pltpu.VMEM((num_b,), jnp.int32),
            pltpu.VMEM((rows_per_w, num_visits), jnp.float32),
            pltpu.SemaphoreType.DMA,
        ],
        compiler_params=pltpu.CompilerParams(needs_layout_passes=False),
    )
    def k(vb_hbm, out_hbm, vb, buf, sem):
        wid = lax.axis_index("s") * 2 + lax.axis_index("c")
        vb_copy = pltpu.async_copy(vb_hbm, vb, sem)

        zeros = jnp.zeros((_LANES,), jnp.float32)
        n_full = num_visits // _LANES
        col_starts = [c * _LANES for c in range(n_full)]
        if num_visits % _LANES:
            col_starts.append(num_visits - _LANES)

        def zero_body(r, carry):
            for c in col_starts:
                buf[r, pl.ds(c, _LANES)] = zeros
            return carry

        lax.fori_loop(0, rows_per_w, zero_body, 0)
        vb_copy.wait()

        base = wid * rows_per_w
        lane = lax.iota(jnp.int32, _LANES)
        ones = jnp.ones((_LANES,), jnp.float32)

        def group_body(g, carry):
            local = g * _LANES + lane
            r = base + local
            pos = jnp.zeros((_LANES,), jnp.int32)
            s = num_b // 2
            while s >= 1:
                val = plsc.load_gather(vb, [pos + (s - 1)])
                pos = jnp.where(val <= r, pos + s, pos)
                s //= 2
            val = plsc.load_gather(vb, [pos])
            pos = jnp.where(val <= r, pos + 1, pos)
            plsc.store_scatter(buf, [local, pos], ones)
            return carry

        lax.fori_loop(0, groups, group_body, 0)

        pltpu.sync_copy(buf, out_hbm.at[pl.ds(wid * rows_per_w, rows_per_w)])

    return k(visit_boundaries)


def kernel(X, visit_boundaries):
    seq_len = X.shape[0]
    num_visits = visit_boundaries.shape[0] + 1
    return _build_h(
        visit_boundaries.astype(jnp.int32), seq_len=seq_len, num_visits=num_visits
    ).astype(X.dtype)
